# baseline (speedup 1.0000x reference)
import functools

import jax
import jax.numpy as jnp
from jax import lax
from jax.experimental import pallas as pl
from jax.experimental.pallas import tpu as pltpu
from jax.experimental.pallas import tpu_sc as plsc

_N_NODES = 10000
_N_EDGES = 160000
_D = 256
_N_REL = 200

_L = 16
_NW = 32
_ROWS_W = 312
_EXTRA_LO = _NW * _ROWS_W
_DUMP = _ROWS_W + _L
_ACC_ROWS = 336
_SCAN = 256
_N_CHUNKS = _N_EDGES // _SCAN
_FIRE = 64
_CAP = 320


def _mp_body(ent_hbm, rel_hbm, src_hbm, dst_hbm, type_hbm, norm_hbm,
             out_hbm, dstm, srcm, typem, normm, row_f, src_f, type_f, norm_f,
             gidx, tidx, xj_v, rel_c, acc_v, sem_a, sem_b):
    c = lax.axis_index("c")
    s = lax.axis_index("s")
    w = s * 2 + c
    lo = w * _ROWS_W
    is_last = w == _NW - 1

    iota = lax.iota(jnp.int32, _L)
    zv = jnp.zeros((_L,), jnp.float32)

    def _zero_row(r, carry):
        for dblk in range(_D // _L):
            acc_v[r, pl.ds(dblk * _L, _L)] = zv
        return carry

    lax.fori_loop(0, _ACC_ROWS, _zero_row, 0)

    def _fire(ptr):
        for g in range(_FIRE // _L):
            gidx[pl.ds(g * _L, _L)] = src_f[pl.ds(g * _L, _L)]
            tidx[pl.ds(g * _L, _L)] = type_f[pl.ds(g * _L, _L)]
        cp_x = pltpu.async_copy(ent_hbm.at[gidx], xj_v, sem_a)
        cp_r = pltpu.async_copy(rel_hbm.at[tidx], rel_c, sem_b)
        cp_x.wait()
        cp_r.wait()

        def _edge(e, ecarry):
            nrm = norm_f[pl.ds(e, _L)][0]
            row = row_f[pl.ds(e, _L)][0]
            nv = jnp.full((_L,), nrm, jnp.float32)
            for dblk in range(_D // _L):
                x = xj_v[e, pl.ds(dblk * _L, _L)]
                r = rel_c[e, pl.ds(dblk * _L, _L)]
                plsc.addupdate(acc_v.at[row, pl.ds(dblk * _L, _L)],
                               (x - r) * nv)
            return ecarry

        lax.fori_loop(0, _FIRE, _edge, 0)

        for g in range((_CAP - _FIRE) // _L):
            sl_from = pl.ds(_FIRE + g * _L, _L)
            sl_to = pl.ds(g * _L, _L)
            row_f[sl_to] = row_f[sl_from]
            src_f[sl_to] = src_f[sl_from]
            type_f[sl_to] = type_f[sl_from]
            norm_f[sl_to] = norm_f[sl_from]
        return ptr - _FIRE

    def _chunk(k, ptr):
        base = k * _SCAN
        pltpu.sync_copy(dst_hbm.at[pl.ds(base, _SCAN)], dstm)
        pltpu.sync_copy(src_hbm.at[pl.ds(base, _SCAN)], srcm)
        pltpu.sync_copy(type_hbm.at[pl.ds(base, _SCAN)], typem)
        pltpu.sync_copy(norm_hbm.at[pl.ds(base, _SCAN)], normm)

        for g in range(_SCAN // _L):
            sl = pl.ds(g * _L, _L)
            d = lax.rem(dstm[sl], jnp.int32(_N_NODES))
            in_main = jnp.logical_and(d >= lo, d < lo + _ROWS_W)
            wv = jnp.full((_L,), w, jnp.int32)
            in_extra = jnp.logical_and(wv == _NW - 1, d >= _EXTRA_LO)
            owned = jnp.logical_or(in_main, in_extra)
            row = jnp.where(in_main, d - lo, d - _EXTRA_LO + _ROWS_W)
            ki = owned.astype(jnp.int32)
            pos = ptr + plsc.cumsum(ki) - 1
            pos = jnp.where(owned, pos, _CAP - 1)
            plsc.store_scatter(row_f, [pos], row)
            plsc.store_scatter(src_f, [pos], srcm[sl])
            plsc.store_scatter(type_f, [pos], typem[sl])
            plsc.store_scatter(norm_f, [pos], normm[sl])
            ptr = ptr + jnp.sum(ki)

        return lax.while_loop(lambda p: p >= _FIRE, _fire, ptr)

    ptr = lax.fori_loop(0, _N_CHUNKS, _chunk, 0)

    pv = jnp.full((_L,), ptr, jnp.int32)
    for g in range(_FIRE // _L):
        sl = pl.ds(g * _L, _L)
        m = (iota + g * _L) < pv
        row_f[sl] = jnp.where(m, row_f[sl], _DUMP)
        src_f[sl] = jnp.where(m, src_f[sl], 0)
        type_f[sl] = jnp.where(m, type_f[sl], 0)
        norm_f[sl] = jnp.where(m, norm_f[sl], jnp.float32(0.0))
    _fire(jnp.int32(_FIRE))

    pltpu.sync_copy(acc_v.at[pl.ds(0, _ROWS_W)],
                    out_hbm.at[pl.ds(lo, _ROWS_W)])

    @pl.when(is_last)
    def _():
        pltpu.sync_copy(acc_v.at[pl.ds(_ROWS_W, _L)],
                        out_hbm.at[pl.ds(_EXTRA_LO, _L)])


_mp_kernel = functools.partial(
    pl.kernel,
    out_type=jax.ShapeDtypeStruct((_N_NODES, _D), jnp.float32),
    mesh=plsc.VectorSubcoreMesh(core_axis_name="c", subcore_axis_name="s"),
    compiler_params=pltpu.CompilerParams(needs_layout_passes=False),
    scratch_types=[
        pltpu.VMEM((_SCAN,), jnp.int32),
        pltpu.VMEM((_SCAN,), jnp.int32),
        pltpu.VMEM((_SCAN,), jnp.int32),
        pltpu.VMEM((_SCAN,), jnp.float32),
        pltpu.VMEM((_CAP,), jnp.int32),
        pltpu.VMEM((_CAP,), jnp.int32),
        pltpu.VMEM((_CAP,), jnp.int32),
        pltpu.VMEM((_CAP,), jnp.float32),
        pltpu.VMEM((_FIRE,), jnp.int32),
        pltpu.VMEM((_FIRE,), jnp.int32),
        pltpu.VMEM((_FIRE, _D), jnp.float32),
        pltpu.VMEM((_FIRE, _D), jnp.float32),
        pltpu.VMEM((_ACC_ROWS, _D), jnp.float32),
        pltpu.SemaphoreType.DMA,
        pltpu.SemaphoreType.DMA,
    ],
)(_mp_body)


def kernel(edge_index, edge_type, ent_embed, rel_embed, edge_norm):
    src = edge_index[1]
    dst = edge_index[0]
    return _mp_kernel(ent_embed, rel_embed, src, dst, edge_type, edge_norm)


# speedup vs baseline: 1.8772x; 1.8772x over previous
import functools

import jax
import jax.numpy as jnp
from jax import lax
from jax.experimental import pallas as pl
from jax.experimental.pallas import tpu as pltpu
from jax.experimental.pallas import tpu_sc as plsc

_N_NODES = 10000
_N_EDGES = 160000
_D = 256
_N_REL = 200

_L = 16
_NW = 32
_ROWS_W = 312
_EXTRA_LO = _NW * _ROWS_W
_DUMP = _ROWS_W + _L
_ACC_ROWS = 336
_SCAN = 640
_N_CHUNKS = _N_EDGES // _SCAN
_FIRE = 64
_CAP = 704


def _mp_body(ent_hbm, rel_hbm, meta_hbm,
             out_hbm, meta_a, meta_b, row_f, src_f, type_f, norm_f,
             gidx, tidx, xj_v, rel_c, acc_v, sem_a, sem_b, sem_x, sem_r):
    c = lax.axis_index("c")
    s = lax.axis_index("s")
    w = s * 2 + c
    lo = w * _ROWS_W

    iota = lax.iota(jnp.int32, _L)
    zv = jnp.zeros((_L,), jnp.float32)

    def _zero_row(r, carry):
        for dblk in range(_D // _L):
            acc_v[r, pl.ds(dblk * _L, _L)] = zv
        return carry

    lax.fori_loop(0, _ACC_ROWS, _zero_row, 0)

    def _fire(ptr):
        for g in range(_FIRE // _L):
            gidx[pl.ds(g * _L, _L)] = src_f[pl.ds(g * _L, _L)]
            tidx[pl.ds(g * _L, _L)] = type_f[pl.ds(g * _L, _L)]
        cp_x = pltpu.async_copy(ent_hbm.at[gidx], xj_v, sem_x)
        cp_r = pltpu.async_copy(rel_hbm.at[tidx], rel_c, sem_r)
        cp_x.wait()
        cp_r.wait()

        def _edge(e, ecarry):
            nrm = norm_f[pl.ds(e, _L)][0]
            row = row_f[pl.ds(e, _L)][0]
            nv = jnp.full((_L,), nrm, jnp.float32)
            for dblk in range(_D // _L):
                x = xj_v[e, pl.ds(dblk * _L, _L)]
                r = rel_c[e, pl.ds(dblk * _L, _L)]
                plsc.addupdate(acc_v.at[row, pl.ds(dblk * _L, _L)],
                               (x - r) * nv)
            return ecarry

        lax.fori_loop(0, _FIRE, _edge, 0)

        def _shift(g, scarry):
            sl_from = pl.ds(_FIRE + g * _L, _L)
            sl_to = pl.ds(g * _L, _L)
            row_f[sl_to] = row_f[sl_from]
            src_f[sl_to] = src_f[sl_from]
            type_f[sl_to] = type_f[sl_from]
            norm_f[sl_to] = norm_f[sl_from]
            return scarry

        lax.fori_loop(0, (_CAP - _FIRE) // _L, _shift, 0)
        return ptr - _FIRE

    def _scan_buf(buf, ptr):
        def _grp(g, ptr):
            sl = pl.ds(g * _L, _L)
            d = lax.rem(buf[0, sl], jnp.int32(_N_NODES))
            in_main = jnp.logical_and(d >= lo, d < lo + _ROWS_W)
            wv = jnp.full((_L,), w, jnp.int32)
            in_extra = jnp.logical_and(wv == _NW - 1, d >= _EXTRA_LO)
            owned = jnp.logical_or(in_main, in_extra)
            row = jnp.where(in_main, d - lo, d - _EXTRA_LO + _ROWS_W)
            ki = jnp.where(owned, 1, 0)
            cs = plsc.cumsum(ki)
            pos = ptr + cs - 1
            pos = jnp.where(owned, pos, _CAP - 1)
            plsc.store_scatter(row_f, [pos], row)
            plsc.store_scatter(src_f, [pos], buf[1, sl])
            plsc.store_scatter(type_f, [pos], buf[2, sl])
            plsc.store_scatter(norm_f, [pos],
                               plsc.bitcast(buf[3, sl], jnp.float32))
            return ptr + cs[_L - 1]

        ptr = lax.fori_loop(0, _SCAN // _L, _grp, ptr)
        return lax.while_loop(lambda p: p >= _FIRE, _fire, ptr)

    def _issue(k, buf, sem):
        return pltpu.async_copy(meta_hbm.at[:, pl.ds(k * _SCAN, _SCAN)],
                                buf, sem)

    _issue(0, meta_a, sem_a)

    def _pair(j, ptr):
        pltpu.make_async_copy(meta_hbm.at[:, pl.ds(0, _SCAN)], meta_a, sem_a).wait()
        _issue(2 * j + 1, meta_b, sem_b)
        ptr = _scan_buf(meta_a, ptr)
        pltpu.make_async_copy(meta_hbm.at[:, pl.ds(0, _SCAN)], meta_b, sem_b).wait()
        _issue(jnp.minimum(2 * j + 2, _N_CHUNKS - 1), meta_a, sem_a)
        ptr = _scan_buf(meta_b, ptr)
        return ptr

    ptr = lax.fori_loop(0, _N_CHUNKS // 2, _pair, 0)
    pltpu.make_async_copy(meta_hbm.at[:, pl.ds(0, _SCAN)], meta_a, sem_a).wait()

    pv = jnp.full((_L,), ptr, jnp.int32)
    for g in range(_FIRE // _L):
        sl = pl.ds(g * _L, _L)
        m = (iota + g * _L) < pv
        row_f[sl] = jnp.where(m, row_f[sl], _DUMP)
        src_f[sl] = jnp.where(m, src_f[sl], 0)
        type_f[sl] = jnp.where(m, type_f[sl], 0)
        norm_f[sl] = jnp.where(m, norm_f[sl], jnp.float32(0.0))
    _fire(jnp.int32(_FIRE))

    pltpu.sync_copy(acc_v.at[pl.ds(0, _ROWS_W)],
                    out_hbm.at[pl.ds(lo, _ROWS_W)])

    @pl.when(w == _NW - 1)
    def _():
        pltpu.sync_copy(acc_v.at[pl.ds(_ROWS_W, _L)],
                        out_hbm.at[pl.ds(_EXTRA_LO, _L)])


_mp_kernel = functools.partial(
    pl.kernel,
    out_type=jax.ShapeDtypeStruct((_N_NODES, _D), jnp.float32),
    mesh=plsc.VectorSubcoreMesh(core_axis_name="c", subcore_axis_name="s"),
    compiler_params=pltpu.CompilerParams(needs_layout_passes=False),
    scratch_types=[
        pltpu.VMEM((4, _SCAN), jnp.int32),
        pltpu.VMEM((4, _SCAN), jnp.int32),
        pltpu.VMEM((_CAP,), jnp.int32),
        pltpu.VMEM((_CAP,), jnp.int32),
        pltpu.VMEM((_CAP,), jnp.int32),
        pltpu.VMEM((_CAP,), jnp.float32),
        pltpu.VMEM((_FIRE,), jnp.int32),
        pltpu.VMEM((_FIRE,), jnp.int32),
        pltpu.VMEM((_FIRE, _D), jnp.float32),
        pltpu.VMEM((_FIRE, _D), jnp.float32),
        pltpu.VMEM((_ACC_ROWS, _D), jnp.float32),
        pltpu.SemaphoreType.DMA,
        pltpu.SemaphoreType.DMA,
        pltpu.SemaphoreType.DMA,
        pltpu.SemaphoreType.DMA,
    ],
)(_mp_body)


def kernel(edge_index, edge_type, ent_embed, rel_embed, edge_norm):
    src = edge_index[1]
    dst = edge_index[0]
    packed = jnp.stack([dst, src, edge_type,
                        lax.bitcast_convert_type(edge_norm, jnp.int32)])
    return _mp_kernel(ent_embed, rel_embed, packed)


# speedup vs baseline: 2.0321x; 1.0825x over previous
import functools

import jax
import jax.numpy as jnp
from jax import lax
from jax.experimental import pallas as pl
from jax.experimental.pallas import tpu as pltpu
from jax.experimental.pallas import tpu_sc as plsc

_N_NODES = 10000
_N_EDGES = 160000
_D = 256
_N_REL = 200

_L = 16
_NW = 32
_ROWS_W = 312
_EXTRA_LO = _NW * _ROWS_W
_DUMP = _ROWS_W + _L
_ACC_ROWS = 336
_SCAN = 640
_N_CHUNKS = _N_EDGES // _SCAN
_FIRE = 32
_CAP = 704


def _mp_body(ent_hbm, rel_hbm, meta_hbm,
             out_hbm, meta_a, meta_b, row_f, src_f, type_f, norm_f,
             gidx, tidx, pend_row, pend_norm, xj_v, rel_c, acc_v,
             sem_a, sem_b, sem_x, sem_r):
    c = lax.axis_index("c")
    s = lax.axis_index("s")
    w = s * 2 + c
    lo = w * _ROWS_W

    iota = lax.iota(jnp.int32, _L)
    zv = jnp.zeros((_L,), jnp.float32)

    def _zero_row(r, carry):
        for dblk in range(_D // _L):
            acc_v[r, pl.ds(dblk * _L, _L)] = zv
        return carry

    lax.fori_loop(0, _ACC_ROWS, _zero_row, 0)

    def _wait_prev():
        pltpu.make_async_copy(ent_hbm.at[pl.ds(0, _FIRE)],
                              xj_v.at[pl.ds(0, _FIRE)], sem_x).wait()
        pltpu.make_async_copy(rel_hbm.at[pl.ds(0, _FIRE)],
                              rel_c.at[pl.ds(0, _FIRE)], sem_r).wait()

    def _issue_batch(par):
        base = par * _FIRE
        pltpu.async_copy(ent_hbm.at[gidx], xj_v.at[pl.ds(base, _FIRE)], sem_x)
        pltpu.async_copy(rel_hbm.at[tidx], rel_c.at[pl.ds(base, _FIRE)], sem_r)

    def _compute_batch(par):
        base = par * _FIRE

        def _edge(e, ecarry):
            nrm = pend_norm[pl.ds(base + e, _L)][0]
            row = pend_row[pl.ds(base + e, _L)][0]
            nv = jnp.full((_L,), nrm, jnp.float32)
            for dblk in range(_D // _L):
                x = xj_v[base + e, pl.ds(dblk * _L, _L)]
                r = rel_c[base + e, pl.ds(dblk * _L, _L)]
                plsc.addupdate(acc_v.at[row, pl.ds(dblk * _L, _L)],
                               (x - r) * nv)
            return ecarry

        lax.fori_loop(0, _FIRE, _edge, 0)

    def _fire(carry):
        ptr, par = carry
        _wait_prev()
        for g in range(_FIRE // _L):
            sl = pl.ds(g * _L, _L)
            gidx[sl] = src_f[sl]
            tidx[sl] = type_f[sl]
            pend_row[pl.ds(par * _FIRE + g * _L, _L)] = row_f[sl]
            pend_norm[pl.ds(par * _FIRE + g * _L, _L)] = norm_f[sl]
        _issue_batch(par)
        _compute_batch(1 - par)

        def _shift(g, scarry):
            sl_from = pl.ds(_FIRE + g * _L, _L)
            sl_to = pl.ds(g * _L, _L)
            row_f[sl_to] = row_f[sl_from]
            src_f[sl_to] = src_f[sl_from]
            type_f[sl_to] = type_f[sl_from]
            norm_f[sl_to] = norm_f[sl_from]
            return scarry

        lax.fori_loop(0, (_CAP - _FIRE) // _L, _shift, 0)
        return (ptr - _FIRE, 1 - par)

    def _scan_buf(buf, ptr, par):
        def _grp(g, ptr):
            sl = pl.ds(g * _L, _L)
            d = lax.rem(buf[0, sl], jnp.int32(_N_NODES))
            in_main = jnp.logical_and(d >= lo, d < lo + _ROWS_W)
            wv = jnp.full((_L,), w, jnp.int32)
            in_extra = jnp.logical_and(wv == _NW - 1, d >= _EXTRA_LO)
            owned = jnp.logical_or(in_main, in_extra)
            row = jnp.where(in_main, d - lo, d - _EXTRA_LO + _ROWS_W)
            ki = jnp.where(owned, 1, 0)
            cs = plsc.cumsum(ki)
            pos = ptr + cs - 1
            pos = jnp.where(owned, pos, _CAP - 1)
            plsc.store_scatter(row_f, [pos], row)
            plsc.store_scatter(src_f, [pos], buf[1, sl])
            plsc.store_scatter(type_f, [pos], buf[2, sl])
            plsc.store_scatter(norm_f, [pos],
                               plsc.bitcast(buf[3, sl], jnp.float32))
            return ptr + cs[_L - 1]

        ptr = lax.fori_loop(0, _SCAN // _L, _grp, ptr)
        return lax.while_loop(lambda cpp: cpp[0] >= _FIRE, _fire, (ptr, par))

    def _issue(k, buf, sem):
        return pltpu.async_copy(meta_hbm.at[:, pl.ds(k * _SCAN, _SCAN)],
                                buf, sem)

    _issue(0, meta_a, sem_a)

    for g in range(_FIRE // _L):
        sl = pl.ds(g * _L, _L)
        gidx[sl] = jnp.full((_L,), 0, jnp.int32)
        tidx[sl] = jnp.full((_L,), 0, jnp.int32)
        pend_row[sl] = jnp.full((_L,), _DUMP, jnp.int32)
        pend_norm[sl] = jnp.full((_L,), 0.0, jnp.float32)
    _issue_batch(0)

    def _pair(j, carry):
        ptr, par = carry
        pltpu.make_async_copy(meta_hbm.at[:, pl.ds(0, _SCAN)], meta_a, sem_a).wait()
        _issue(2 * j + 1, meta_b, sem_b)
        ptr, par = _scan_buf(meta_a, ptr, par)
        pltpu.make_async_copy(meta_hbm.at[:, pl.ds(0, _SCAN)], meta_b, sem_b).wait()
        _issue(jnp.minimum(2 * j + 2, _N_CHUNKS - 1), meta_a, sem_a)
        ptr, par = _scan_buf(meta_b, ptr, par)
        return (ptr, par)

    ptr, par = lax.fori_loop(0, _N_CHUNKS // 2, _pair, (0, 1))
    pltpu.make_async_copy(meta_hbm.at[:, pl.ds(0, _SCAN)], meta_a, sem_a).wait()

    pv = jnp.full((_L,), ptr, jnp.int32)
    for g in range(_FIRE // _L):
        sl = pl.ds(g * _L, _L)
        m = (iota + g * _L) < pv
        row_f[sl] = jnp.where(m, row_f[sl], _DUMP)
        src_f[sl] = jnp.where(m, src_f[sl], 0)
        type_f[sl] = jnp.where(m, type_f[sl], 0)
        norm_f[sl] = jnp.where(m, norm_f[sl], jnp.float32(0.0))
    ptr, par = _fire((jnp.int32(_FIRE), par))
    _wait_prev()
    _compute_batch(1 - par)

    pltpu.sync_copy(acc_v.at[pl.ds(0, _ROWS_W)],
                    out_hbm.at[pl.ds(lo, _ROWS_W)])

    @pl.when(w == _NW - 1)
    def _():
        pltpu.sync_copy(acc_v.at[pl.ds(_ROWS_W, _L)],
                        out_hbm.at[pl.ds(_EXTRA_LO, _L)])


_mp_kernel = functools.partial(
    pl.kernel,
    out_type=jax.ShapeDtypeStruct((_N_NODES, _D), jnp.float32),
    mesh=plsc.VectorSubcoreMesh(core_axis_name="c", subcore_axis_name="s"),
    compiler_params=pltpu.CompilerParams(needs_layout_passes=False),
    scratch_types=[
        pltpu.VMEM((4, _SCAN), jnp.int32),
        pltpu.VMEM((4, _SCAN), jnp.int32),
        pltpu.VMEM((_CAP,), jnp.int32),
        pltpu.VMEM((_CAP,), jnp.int32),
        pltpu.VMEM((_CAP,), jnp.int32),
        pltpu.VMEM((_CAP,), jnp.float32),
        pltpu.VMEM((_FIRE,), jnp.int32),
        pltpu.VMEM((_FIRE,), jnp.int32),
        pltpu.VMEM((80,), jnp.int32),
        pltpu.VMEM((80,), jnp.float32),
        pltpu.VMEM((2 * _FIRE, _D), jnp.float32),
        pltpu.VMEM((2 * _FIRE, _D), jnp.float32),
        pltpu.VMEM((_ACC_ROWS, _D), jnp.float32),
        pltpu.SemaphoreType.DMA,
        pltpu.SemaphoreType.DMA,
        pltpu.SemaphoreType.DMA,
        pltpu.SemaphoreType.DMA,
    ],
)(_mp_body)


def kernel(edge_index, edge_type, ent_embed, rel_embed, edge_norm):
    src = edge_index[1]
    dst = edge_index[0]
    packed = jnp.stack([dst, src, edge_type,
                        lax.bitcast_convert_type(edge_norm, jnp.int32)])
    return _mp_kernel(ent_embed, rel_embed, packed)


# speedup vs baseline: 2.0322x; 1.0001x over previous
import functools

import jax
import jax.numpy as jnp
from jax import lax
from jax.experimental import pallas as pl
from jax.experimental.pallas import tpu as pltpu
from jax.experimental.pallas import tpu_sc as plsc

_N_NODES = 10000
_N_EDGES = 160000
_D = 256
_N_REL = 200

_L = 16
_NW = 32
_ROWS_W = 312
_EXTRA_LO = _NW * _ROWS_W
_DUMP = _ROWS_W + _L
_ACC_ROWS = 336
_SCAN = 640
_N_CHUNKS = _N_EDGES // _SCAN
_FIRE = 32
_CAP = 704


def _mp_body(ent_hbm, rel_hbm, meta_hbm,
             out_hbm, meta_a, meta_b, row_f, src_f, type_f, norm_f,
             gidx, tidx, pend_row, pend_norm, xj_v, rel_c, acc_v,
             sem_a, sem_b, sem_x, sem_r):
    c = lax.axis_index("c")
    s = lax.axis_index("s")
    w = s * 2 + c
    lo = w * _ROWS_W

    iota = lax.iota(jnp.int32, _L)
    zv = jnp.zeros((_L,), jnp.float32)

    def _zero_row(r, carry):
        for dblk in range(_D // _L):
            acc_v[r, pl.ds(dblk * _L, _L)] = zv
        return carry

    lax.fori_loop(0, _ACC_ROWS, _zero_row, 0)

    def _wait_prev():
        pltpu.make_async_copy(ent_hbm.at[pl.ds(0, _FIRE)],
                              xj_v.at[pl.ds(0, _FIRE)], sem_x).wait()
        pltpu.make_async_copy(rel_hbm.at[pl.ds(0, _FIRE)],
                              rel_c.at[pl.ds(0, _FIRE)], sem_r).wait()

    def _issue_batch(par):
        base = par * _FIRE
        pltpu.async_copy(ent_hbm.at[gidx], xj_v.at[pl.ds(base, _FIRE)], sem_x)
        pltpu.async_copy(rel_hbm.at[tidx], rel_c.at[pl.ds(base, _FIRE)], sem_r)

    def _compute_batch(par):
        base = par * _FIRE

        def _edge(e, ecarry):
            nrm = pend_norm[pl.ds(base + e, _L)][0]
            row = pend_row[pl.ds(base + e, _L)][0]
            nv = jnp.full((_L,), nrm, jnp.float32)
            for dblk in range(_D // _L):
                x = xj_v[base + e, pl.ds(dblk * _L, _L)]
                r = rel_c[base + e, pl.ds(dblk * _L, _L)]
                plsc.addupdate(acc_v.at[row, pl.ds(dblk * _L, _L)],
                               (x - r) * nv)
            return ecarry

        lax.fori_loop(0, _FIRE, _edge, 0)

    def _fire(carry):
        ptr, par = carry
        _wait_prev()
        for g in range(_FIRE // _L):
            sl = pl.ds(g * _L, _L)
            gidx[sl] = src_f[sl]
            tidx[sl] = type_f[sl]
            pend_row[pl.ds(par * _FIRE + g * _L, _L)] = row_f[sl]
            pend_norm[pl.ds(par * _FIRE + g * _L, _L)] = norm_f[sl]
        _issue_batch(par)
        _compute_batch(1 - par)

        def _shift(g, scarry):
            sl_from = pl.ds(_FIRE + g * _L, _L)
            sl_to = pl.ds(g * _L, _L)
            row_f[sl_to] = row_f[sl_from]
            src_f[sl_to] = src_f[sl_from]
            type_f[sl_to] = type_f[sl_from]
            norm_f[sl_to] = norm_f[sl_from]
            return scarry

        lax.fori_loop(0, (_CAP - _FIRE) // _L, _shift, 0)
        return (ptr - _FIRE, 1 - par)

    def _scan_buf(buf, ptr, par):
        def _grp(g, ptr):
            sl = pl.ds(g * _L, _L)
            d = lax.rem(buf[0, sl], jnp.int32(_N_NODES))
            in_main = jnp.logical_and(d >= lo, d < lo + _ROWS_W)
            wv = jnp.full((_L,), w, jnp.int32)
            in_extra = jnp.logical_and(wv == _NW - 1, d >= _EXTRA_LO)
            owned = jnp.logical_or(in_main, in_extra)
            row = jnp.where(in_main, d - lo, d - _EXTRA_LO + _ROWS_W)
            ki = jnp.where(owned, 1, 0)
            cs = plsc.cumsum(ki)
            pos = ptr + cs - 1
            pos = jnp.where(owned, pos, _CAP - 1)
            plsc.store_scatter(row_f, [pos], row)
            plsc.store_scatter(src_f, [pos], buf[1, sl])
            plsc.store_scatter(type_f, [pos], buf[2, sl])
            plsc.store_scatter(norm_f, [pos],
                               plsc.bitcast(buf[3, sl], jnp.float32))
            return ptr + plsc.all_reduce_population_count(owned)[0]

        ptr = lax.fori_loop(0, _SCAN // _L, _grp, ptr)
        return lax.while_loop(lambda cpp: cpp[0] >= _FIRE, _fire, (ptr, par))

    def _issue(k, buf, sem):
        return pltpu.async_copy(meta_hbm.at[:, pl.ds(k * _SCAN, _SCAN)],
                                buf, sem)

    _issue(0, meta_a, sem_a)

    for g in range(_FIRE // _L):
        sl = pl.ds(g * _L, _L)
        gidx[sl] = jnp.full((_L,), 0, jnp.int32)
        tidx[sl] = jnp.full((_L,), 0, jnp.int32)
        pend_row[sl] = jnp.full((_L,), _DUMP, jnp.int32)
        pend_norm[sl] = jnp.full((_L,), 0.0, jnp.float32)
    _issue_batch(0)

    def _pair(j, carry):
        ptr, par = carry
        pltpu.make_async_copy(meta_hbm.at[:, pl.ds(0, _SCAN)], meta_a, sem_a).wait()
        _issue(2 * j + 1, meta_b, sem_b)
        ptr, par = _scan_buf(meta_a, ptr, par)
        pltpu.make_async_copy(meta_hbm.at[:, pl.ds(0, _SCAN)], meta_b, sem_b).wait()
        _issue(jnp.minimum(2 * j + 2, _N_CHUNKS - 1), meta_a, sem_a)
        ptr, par = _scan_buf(meta_b, ptr, par)
        return (ptr, par)

    ptr, par = lax.fori_loop(0, _N_CHUNKS // 2, _pair, (0, 1))
    pltpu.make_async_copy(meta_hbm.at[:, pl.ds(0, _SCAN)], meta_a, sem_a).wait()

    pv = jnp.full((_L,), ptr, jnp.int32)
    for g in range(_FIRE // _L):
        sl = pl.ds(g * _L, _L)
        m = (iota + g * _L) < pv
        row_f[sl] = jnp.where(m, row_f[sl], _DUMP)
        src_f[sl] = jnp.where(m, src_f[sl], 0)
        type_f[sl] = jnp.where(m, type_f[sl], 0)
        norm_f[sl] = jnp.where(m, norm_f[sl], jnp.float32(0.0))
    ptr, par = _fire((jnp.int32(_FIRE), par))
    _wait_prev()
    _compute_batch(1 - par)

    pltpu.sync_copy(acc_v.at[pl.ds(0, _ROWS_W)],
                    out_hbm.at[pl.ds(lo, _ROWS_W)])

    @pl.when(w == _NW - 1)
    def _():
        pltpu.sync_copy(acc_v.at[pl.ds(_ROWS_W, _L)],
                        out_hbm.at[pl.ds(_EXTRA_LO, _L)])


_mp_kernel = functools.partial(
    pl.kernel,
    out_type=jax.ShapeDtypeStruct((_N_NODES, _D), jnp.float32),
    mesh=plsc.VectorSubcoreMesh(core_axis_name="c", subcore_axis_name="s"),
    compiler_params=pltpu.CompilerParams(needs_layout_passes=False),
    scratch_types=[
        pltpu.VMEM((4, _SCAN), jnp.int32),
        pltpu.VMEM((4, _SCAN), jnp.int32),
        pltpu.VMEM((_CAP,), jnp.int32),
        pltpu.VMEM((_CAP,), jnp.int32),
        pltpu.VMEM((_CAP,), jnp.int32),
        pltpu.VMEM((_CAP,), jnp.float32),
        pltpu.VMEM((_FIRE,), jnp.int32),
        pltpu.VMEM((_FIRE,), jnp.int32),
        pltpu.VMEM((80,), jnp.int32),
        pltpu.VMEM((80,), jnp.float32),
        pltpu.VMEM((2 * _FIRE, _D), jnp.float32),
        pltpu.VMEM((2 * _FIRE, _D), jnp.float32),
        pltpu.VMEM((_ACC_ROWS, _D), jnp.float32),
        pltpu.SemaphoreType.DMA,
        pltpu.SemaphoreType.DMA,
        pltpu.SemaphoreType.DMA,
        pltpu.SemaphoreType.DMA,
    ],
)(_mp_body)


def kernel(edge_index, edge_type, ent_embed, rel_embed, edge_norm):
    src = edge_index[1]
    dst = edge_index[0]
    packed = jnp.stack([dst, src, edge_type,
                        lax.bitcast_convert_type(edge_norm, jnp.int32)])
    return _mp_kernel(ent_embed, rel_embed, packed)


# speedup vs baseline: 2.0619x; 1.0146x over previous
import functools

import jax
import jax.numpy as jnp
from jax import lax
from jax.experimental import pallas as pl
from jax.experimental.pallas import tpu as pltpu
from jax.experimental.pallas import tpu_sc as plsc

_N_NODES = 10000
_N_EDGES = 160000
_D = 256
_N_REL = 200

_L = 16
_NW = 32
_ROWS_W = 312
_EXTRA_LO = _NW * _ROWS_W
_DUMP = _ROWS_W + _L
_ACC_ROWS = 336
_SCAN = 640
_N_CHUNKS = _N_EDGES // _SCAN
_FIRE = 32
_CAP = 704


def _mp_body(ent_hbm, rel_hbm, meta_hbm,
             out_hbm, meta_a, meta_b, row_f, src_f, type_f, norm_f,
             gidx, tidx, pend_row, pend_norm, xj_v, rel_c, acc_v,
             sem_a, sem_b, sem_x, sem_r):
    c = lax.axis_index("c")
    s = lax.axis_index("s")
    w = s * 2 + c
    lo = w * _ROWS_W

    iota = lax.iota(jnp.int32, _L)
    zv = jnp.zeros((_L,), jnp.float32)

    def _zero_row(r, carry):
        for dblk in range(_D // _L):
            acc_v[r, pl.ds(dblk * _L, _L)] = zv
        return carry

    lax.fori_loop(0, _ACC_ROWS, _zero_row, 0)

    def _wait_prev():
        pltpu.make_async_copy(ent_hbm.at[pl.ds(0, _FIRE)],
                              xj_v.at[pl.ds(0, _FIRE)], sem_x).wait()
        pltpu.make_async_copy(rel_hbm.at[pl.ds(0, _FIRE)],
                              rel_c.at[pl.ds(0, _FIRE)], sem_r).wait()

    def _issue_batch(par):
        base = par * _FIRE
        pltpu.async_copy(ent_hbm.at[gidx], xj_v.at[pl.ds(base, _FIRE)], sem_x)
        pltpu.async_copy(rel_hbm.at[tidx], rel_c.at[pl.ds(base, _FIRE)], sem_r)

    def _compute_batch(par):
        base = par * _FIRE

        def _edge(e, ecarry):
            nrm = pend_norm[pl.ds(base + e, _L)][0]
            row = pend_row[pl.ds(base + e, _L)][0]
            nv = jnp.full((_L,), nrm, jnp.float32)
            for dblk in range(_D // _L):
                x = xj_v[base + e, pl.ds(dblk * _L, _L)]
                r = rel_c[base + e, pl.ds(dblk * _L, _L)]
                plsc.addupdate(acc_v.at[row, pl.ds(dblk * _L, _L)],
                               (x - r) * nv)
            return ecarry

        lax.fori_loop(0, _FIRE, _edge, 0)

    def _fire(carry):
        ptr, par = carry
        _wait_prev()
        for g in range(_FIRE // _L):
            sl = pl.ds(g * _L, _L)
            gidx[sl] = src_f[sl]
            tidx[sl] = type_f[sl]
            pend_row[pl.ds(par * _FIRE + g * _L, _L)] = row_f[sl]
            pend_norm[pl.ds(par * _FIRE + g * _L, _L)] = norm_f[sl]
        _issue_batch(par)
        _compute_batch(1 - par)

        def _shift(g, scarry):
            sl_from = pl.ds(_FIRE + g * _L, _L)
            sl_to = pl.ds(g * _L, _L)
            row_f[sl_to] = row_f[sl_from]
            src_f[sl_to] = src_f[sl_from]
            type_f[sl_to] = type_f[sl_from]
            norm_f[sl_to] = norm_f[sl_from]
            return scarry

        nsh = lax.shift_right_logical(ptr - _FIRE + _L - 1, 4)
        lax.fori_loop(0, nsh, _shift, 0)
        return (ptr - _FIRE, 1 - par)

    def _scan_buf(buf, ptr, par):
        def _grp(g, ptr):
            sl = pl.ds(g * _L, _L)
            d = lax.rem(buf[0, sl], jnp.int32(_N_NODES))
            in_main = jnp.logical_and(d >= lo, d < lo + _ROWS_W)
            wv = jnp.full((_L,), w, jnp.int32)
            in_extra = jnp.logical_and(wv == _NW - 1, d >= _EXTRA_LO)
            owned = jnp.logical_or(in_main, in_extra)
            row = jnp.where(in_main, d - lo, d - _EXTRA_LO + _ROWS_W)
            ki = jnp.where(owned, 1, 0)
            cs = plsc.cumsum(ki)
            pos = ptr + cs - 1
            pos = jnp.where(owned, pos, _CAP - 1)
            plsc.store_scatter(row_f, [pos], row)
            plsc.store_scatter(src_f, [pos], buf[1, sl])
            plsc.store_scatter(type_f, [pos], buf[2, sl])
            plsc.store_scatter(norm_f, [pos],
                               plsc.bitcast(buf[3, sl], jnp.float32))
            return ptr + plsc.all_reduce_population_count(owned)[0]

        ptr = lax.fori_loop(0, _SCAN // _L, _grp, ptr)
        return lax.while_loop(lambda cpp: cpp[0] >= _FIRE, _fire, (ptr, par))

    def _issue(k, buf, sem):
        return pltpu.async_copy(meta_hbm.at[:, pl.ds(k * _SCAN, _SCAN)],
                                buf, sem)

    _issue(0, meta_a, sem_a)

    for g in range(_FIRE // _L):
        sl = pl.ds(g * _L, _L)
        gidx[sl] = jnp.full((_L,), 0, jnp.int32)
        tidx[sl] = jnp.full((_L,), 0, jnp.int32)
        pend_row[sl] = jnp.full((_L,), _DUMP, jnp.int32)
        pend_norm[sl] = jnp.full((_L,), 0.0, jnp.float32)
    _issue_batch(0)

    def _pair(j, carry):
        ptr, par = carry
        pltpu.make_async_copy(meta_hbm.at[:, pl.ds(0, _SCAN)], meta_a, sem_a).wait()
        _issue(2 * j + 1, meta_b, sem_b)
        ptr, par = _scan_buf(meta_a, ptr, par)
        pltpu.make_async_copy(meta_hbm.at[:, pl.ds(0, _SCAN)], meta_b, sem_b).wait()
        _issue(jnp.minimum(2 * j + 2, _N_CHUNKS - 1), meta_a, sem_a)
        ptr, par = _scan_buf(meta_b, ptr, par)
        return (ptr, par)

    ptr, par = lax.fori_loop(0, _N_CHUNKS // 2, _pair, (0, 1))
    pltpu.make_async_copy(meta_hbm.at[:, pl.ds(0, _SCAN)], meta_a, sem_a).wait()

    pv = jnp.full((_L,), ptr, jnp.int32)
    for g in range(_FIRE // _L):
        sl = pl.ds(g * _L, _L)
        m = (iota + g * _L) < pv
        row_f[sl] = jnp.where(m, row_f[sl], _DUMP)
        src_f[sl] = jnp.where(m, src_f[sl], 0)
        type_f[sl] = jnp.where(m, type_f[sl], 0)
        norm_f[sl] = jnp.where(m, norm_f[sl], jnp.float32(0.0))
    ptr, par = _fire((jnp.int32(_FIRE), par))
    _wait_prev()
    _compute_batch(1 - par)

    pltpu.sync_copy(acc_v.at[pl.ds(0, _ROWS_W)],
                    out_hbm.at[pl.ds(lo, _ROWS_W)])

    @pl.when(w == _NW - 1)
    def _():
        pltpu.sync_copy(acc_v.at[pl.ds(_ROWS_W, _L)],
                        out_hbm.at[pl.ds(_EXTRA_LO, _L)])


_mp_kernel = functools.partial(
    pl.kernel,
    out_type=jax.ShapeDtypeStruct((_N_NODES, _D), jnp.float32),
    mesh=plsc.VectorSubcoreMesh(core_axis_name="c", subcore_axis_name="s"),
    compiler_params=pltpu.CompilerParams(needs_layout_passes=False),
    scratch_types=[
        pltpu.VMEM((4, _SCAN), jnp.int32),
        pltpu.VMEM((4, _SCAN), jnp.int32),
        pltpu.VMEM((_CAP,), jnp.int32),
        pltpu.VMEM((_CAP,), jnp.int32),
        pltpu.VMEM((_CAP,), jnp.int32),
        pltpu.VMEM((_CAP,), jnp.float32),
        pltpu.VMEM((_FIRE,), jnp.int32),
        pltpu.VMEM((_FIRE,), jnp.int32),
        pltpu.VMEM((80,), jnp.int32),
        pltpu.VMEM((80,), jnp.float32),
        pltpu.VMEM((2 * _FIRE, _D), jnp.float32),
        pltpu.VMEM((2 * _FIRE, _D), jnp.float32),
        pltpu.VMEM((_ACC_ROWS, _D), jnp.float32),
        pltpu.SemaphoreType.DMA,
        pltpu.SemaphoreType.DMA,
        pltpu.SemaphoreType.DMA,
        pltpu.SemaphoreType.DMA,
    ],
)(_mp_body)


def kernel(edge_index, edge_type, ent_embed, rel_embed, edge_norm):
    src = edge_index[1]
    dst = edge_index[0]
    packed = jnp.stack([dst, src, edge_type,
                        lax.bitcast_convert_type(edge_norm, jnp.int32)])
    return _mp_kernel(ent_embed, rel_embed, packed)
